# baseline (speedup 1.0000x reference)
import functools

import numpy as np
import jax
import jax.numpy as jnp
from jax import lax
from jax.experimental import pallas as pl
from jax.experimental.pallas import tpu as pltpu
from jax.experimental.pallas import tpu_sc as plsc

_FEATURES_SHAPE = (50, 50)
_STRIDE = 16
_ANCHOR_SIZE = 16
_NUM_GT = 64
_NEG_OVL = 0.4
_POS_OVL = 0.5

_L = 16
_NC = 2
_NS = 16
_NW = _NC * _NS
_N_REAL = 22500
_NPAD = 22528
_CHUNK = _NPAD // _NW
_VPW = _CHUNK // _L
_AUN = 2

_GTLEN = _NUM_GT * _L
_OX1, _OY1, _OX2, _OY2, _OCLS = (i * _GTLEN for i in range(5))
_RAW = _NUM_GT * 5 + 3 + 5
_OIMH_RAW = _NUM_GT * 5
_OIMW_RAW = _NUM_GT * 5 + 1

_LN2 = 0.6931471805599453
_SQRT2 = float(np.float32(np.sqrt(2.0)))


def _gen_anchors(base_size=16):
    ratios = np.array([0.5, 1.0, 2.0])
    scales = np.array([8.0, 16.0, 32.0])
    base = np.array([0.0, 0.0, base_size - 1.0, base_size - 1.0])
    w = base[2] - base[0] + 1.0
    h = base[3] - base[1] + 1.0
    x_ctr = base[0] + 0.5 * (w - 1.0)
    y_ctr = base[1] + 0.5 * (h - 1.0)
    size = w * h
    size_ratios = size / ratios
    ws = np.round(np.sqrt(size_ratios))
    hs = np.round(ws * ratios)

    def _mk(ws, hs, x_ctr, y_ctr):
        ws = ws[:, None]
        hs = hs[:, None]
        return np.hstack([x_ctr - 0.5 * (ws - 1.0), y_ctr - 0.5 * (hs - 1.0),
                          x_ctr + 0.5 * (ws - 1.0), y_ctr + 0.5 * (hs - 1.0)])

    ratio_anchors = _mk(ws, hs, x_ctr, y_ctr)
    out = []
    for i in range(ratio_anchors.shape[0]):
        a = ratio_anchors[i]
        w = a[2] - a[0] + 1.0
        h = a[3] - a[1] + 1.0
        x_ctr = a[0] + 0.5 * (w - 1.0)
        y_ctr = a[1] + 0.5 * (h - 1.0)
        out.append(_mk(w * scales, h * scales, x_ctr, y_ctr))
    return np.vstack(out).astype(np.float32)


def _shift_anchors(shape, stride, anchors):
    sx = np.arange(shape[1]) * stride
    sy = np.arange(shape[0]) * stride
    SX, SY = np.meshgrid(sx, sy)
    shifts = np.stack([SX.ravel(), SY.ravel(), SX.ravel(), SY.ravel()], axis=1)
    return (anchors[None, :, :] + shifts[:, None, :]).reshape(-1, 4).astype(np.float32)


_ANCHORS_NP = _shift_anchors(_FEATURES_SHAPE, _STRIDE, _gen_anchors(_ANCHOR_SIZE))

_APAD = np.zeros((_NPAD, 4), dtype=np.float32)
_APAD[:, :] = np.array([0.0, 0.0, 15.0, 15.0], dtype=np.float32)
_APAD[:_N_REAL] = _ANCHORS_NP
_ABLK = (_APAD.reshape(_NW, _CHUNK, 4).transpose(0, 2, 1)
         .reshape(_NW, 4 * _CHUNK).copy())


def _splat_f(v):
    return jnp.full((_L,), v, jnp.float32)


def _splat_i(v):
    return jnp.full((_L,), v, jnp.int32)


def _log_f32(x):
    bits = lax.bitcast_convert_type(x, jnp.int32)
    e = lax.shift_right_logical(bits, _splat_i(23)) - _splat_i(127)
    mbits = lax.bitwise_or(lax.bitwise_and(bits, _splat_i(0x007FFFFF)),
                           _splat_i(0x3F800000))
    m = lax.bitcast_convert_type(mbits, jnp.float32)
    adj = m > _splat_f(_SQRT2)
    m = jnp.where(adj, m * 0.5, m)
    ef = e.astype(jnp.float32) + jnp.where(adj, _splat_f(1.0), _splat_f(0.0))
    s = (m - 1.0) / (m + 1.0)
    z = s * s
    p = z * (np.float32(1.0 / 3.0) + z * (np.float32(0.2) + z * (
        np.float32(1.0 / 7.0) + z * np.float32(1.0 / 9.0))))
    t2 = s + s
    return ef * np.float32(_LN2) + (t2 + t2 * p)


def _sc_body(raw_h, ablk_h, out_h, raw_v, gtp_v, areab_v, ab_v, ob_v):
    wid = lax.axis_index("c") * _NS + lax.axis_index("s")

    pltpu.sync_copy(raw_h, raw_v)
    pltpu.sync_copy(ablk_h.at[wid], ab_v)

    def rep_step(j, jvec5):
        o = j * _L
        bx1 = plsc.load_gather(raw_v, [jvec5])
        by1 = plsc.load_gather(raw_v, [jvec5 + 1])
        bx2 = plsc.load_gather(raw_v, [jvec5 + 2])
        by2 = plsc.load_gather(raw_v, [jvec5 + 3])
        bcls = plsc.load_gather(raw_v, [jvec5 + 4])
        gtp_v[pl.ds(_OX1 + o, _L)] = bx1
        gtp_v[pl.ds(_OY1 + o, _L)] = by1
        gtp_v[pl.ds(_OX2 + o, _L)] = bx2
        gtp_v[pl.ds(_OY2 + o, _L)] = by2
        gtp_v[pl.ds(_OCLS + o, _L)] = bcls
        areab_v[pl.ds(o, _L)] = (bx2 - bx1 + 1.0) * (by2 - by1 + 1.0)
        return jvec5 + 5

    lax.fori_loop(0, _NUM_GT, rep_step, _splat_i(0))

    imh = plsc.load_gather(raw_v, [_splat_i(_OIMH_RAW)])
    imw = plsc.load_gather(raw_v, [_splat_i(_OIMW_RAW)])
    lane = lax.iota(jnp.int32, _L)

    def anchor_step(v, _):
        ax1 = [None] * _AUN
        ay1 = [None] * _AUN
        ax2 = [None] * _AUN
        ay2 = [None] * _AUN
        area_a = [None] * _AUN
        for k in range(_AUN):
            off = (v * _AUN + k) * _L
            ax1[k] = ab_v[pl.ds(off, _L)]
            ay1[k] = ab_v[pl.ds(_CHUNK + off, _L)]
            ax2[k] = ab_v[pl.ds(2 * _CHUNK + off, _L)]
            ay2[k] = ab_v[pl.ds(3 * _CHUNK + off, _L)]
            area_a[k] = (ax2[k] - ax1[k] + 1.0) * (ay2[k] - ay1[k] + 1.0)

        init = (tuple([_splat_f(-1.0)] * _AUN), tuple([_splat_i(0)] * _AUN),
                _splat_i(0))

        @plsc.parallel_loop(0, _NUM_GT, carry=init, unroll=4)
        def box_loop(j, carry):
            best, bidx, jvec = carry
            best = list(best)
            bidx = list(bidx)
            o = j * _L
            bx1 = gtp_v[pl.ds(_OX1 + o, _L)]
            by1 = gtp_v[pl.ds(_OY1 + o, _L)]
            bx2 = gtp_v[pl.ds(_OX2 + o, _L)]
            by2 = gtp_v[pl.ds(_OY2 + o, _L)]
            areab = areab_v[pl.ds(o, _L)]
            for k in range(_AUN):
                iw = jnp.maximum(
                    jnp.minimum(ax2[k], bx2) - jnp.maximum(ax1[k], bx1) + 1.0, 0.0)
                ih = jnp.maximum(
                    jnp.minimum(ay2[k], by2) - jnp.maximum(ay1[k], by1) + 1.0, 0.0)
                inter = iw * ih
                union = (area_a[k] + areab) - inter
                iou = inter / union
                upd = iou > best[k]
                best[k] = jnp.where(upd, iou, best[k])
                bidx[k] = jnp.where(upd, jvec, bidx[k])
            return tuple(best), tuple(bidx), jvec + 1

        best, bidx, _ = box_loop

        for k in range(_AUN):
            off = (v * _AUN + k) * _L
            gidx = bidx[k] * _L + lane
            bx1 = plsc.load_gather(gtp_v, [gidx + _OX1])
            by1 = plsc.load_gather(gtp_v, [gidx + _OY1])
            bx2 = plsc.load_gather(gtp_v, [gidx + _OX2])
            by2 = plsc.load_gather(gtp_v, [gidx + _OY2])
            bcls = plsc.load_gather(gtp_v, [gidx + _OCLS])

            lab = _splat_f(-1.0)
            lab = jnp.where(best[k] < _NEG_OVL, _splat_f(0.0), lab)
            lab = jnp.where(best[k] >= _POS_OVL, _splat_f(1.0), lab)
            inside = ((ax1[k] >= 0.0) & (ay1[k] >= 0.0)
                      & (ax2[k] < imw) & (ay2[k] < imh))
            lab = jnp.where(inside, lab, _splat_f(-1.0))
            lab = jnp.where(lab == 1.0, bcls, lab)
            ob_v[pl.ds(off, _L)] = lab

            ex_w = ax2[k] - ax1[k] + 1.0
            ex_h = ay2[k] - ay1[k] + 1.0
            gt_w = bx2 - bx1 + 1.0
            gt_h = by2 - by1 + 1.0
            ex_cx = ax1[k] + 0.5 * ex_w
            ex_cy = ay1[k] + 0.5 * ex_h
            gt_cx = bx1 + 0.5 * gt_w
            gt_cy = by1 + 0.5 * gt_h
            ob_v[pl.ds(_CHUNK + off, _L)] = (gt_cx - ex_cx) / ex_w
            ob_v[pl.ds(2 * _CHUNK + off, _L)] = (gt_cy - ex_cy) / ex_h
            ob_v[pl.ds(3 * _CHUNK + off, _L)] = _log_f32(gt_w / ex_w)
            ob_v[pl.ds(4 * _CHUNK + off, _L)] = _log_f32(gt_h / ex_h)
        return 0

    lax.fori_loop(0, _VPW // _AUN, anchor_step, 0)

    pltpu.sync_copy(ob_v, out_h.at[wid])


_sc_call = functools.partial(
    pl.kernel,
    out_type=jax.ShapeDtypeStruct((_NW, 5 * _CHUNK), jnp.float32),
    mesh=plsc.VectorSubcoreMesh(core_axis_name="c", subcore_axis_name="s",
                                num_cores=_NC, num_subcores=_NS),
    compiler_params=pltpu.CompilerParams(needs_layout_passes=False),
    scratch_types=[
        pltpu.VMEM((_RAW,), jnp.float32),
        pltpu.VMEM((5 * _GTLEN,), jnp.float32),
        pltpu.VMEM((_GTLEN,), jnp.float32),
        pltpu.VMEM((4 * _CHUNK,), jnp.float32),
        pltpu.VMEM((5 * _CHUNK,), jnp.float32),
    ],
)(_sc_body)


def kernel(im_info, gt_boxes):
    raw = jnp.concatenate([
        gt_boxes.astype(jnp.float32).reshape(_NUM_GT * 5),
        im_info.astype(jnp.float32).reshape(3),
        jnp.zeros((5,), jnp.float32),
    ])

    out = _sc_call(raw, _ABLK)
    planes = out.reshape(_NW, 5, _CHUNK).transpose(1, 0, 2).reshape(5, _NPAD)
    labels = planes[0, :_N_REAL][None, :]
    targets = planes[1:5, :_N_REAL].T[None, :, :]
    anchors = jnp.asarray(_ANCHORS_NP)[None]
    return labels, targets, anchors


# speedup vs baseline: 1.0373x; 1.0373x over previous
import functools

import numpy as np
import jax
import jax.numpy as jnp
from jax import lax
from jax.experimental import pallas as pl
from jax.experimental.pallas import tpu as pltpu
from jax.experimental.pallas import tpu_sc as plsc

_FEATURES_SHAPE = (50, 50)
_STRIDE = 16
_ANCHOR_SIZE = 16
_NUM_GT = 64
_NEG_OVL = 0.4
_POS_OVL = 0.5

_L = 16
_NC = 2
_NS = 16
_NW = _NC * _NS
_N_REAL = 22500
_NPAD = 22528
_CHUNK = _NPAD // _NW
_VPW = _CHUNK // _L
_AUN = 4

_GTLEN = _NUM_GT * _L
_OX1, _OY1, _OX2, _OY2, _OCLS = (i * _GTLEN for i in range(5))
_RAW = _NUM_GT * 5 + 3 + 5
_OIMH_RAW = _NUM_GT * 5
_OIMW_RAW = _NUM_GT * 5 + 1

_LN2 = 0.6931471805599453
_SQRT2 = float(np.float32(np.sqrt(2.0)))


def _gen_anchors(base_size=16):
    ratios = np.array([0.5, 1.0, 2.0])
    scales = np.array([8.0, 16.0, 32.0])
    base = np.array([0.0, 0.0, base_size - 1.0, base_size - 1.0])
    w = base[2] - base[0] + 1.0
    h = base[3] - base[1] + 1.0
    x_ctr = base[0] + 0.5 * (w - 1.0)
    y_ctr = base[1] + 0.5 * (h - 1.0)
    size = w * h
    size_ratios = size / ratios
    ws = np.round(np.sqrt(size_ratios))
    hs = np.round(ws * ratios)

    def _mk(ws, hs, x_ctr, y_ctr):
        ws = ws[:, None]
        hs = hs[:, None]
        return np.hstack([x_ctr - 0.5 * (ws - 1.0), y_ctr - 0.5 * (hs - 1.0),
                          x_ctr + 0.5 * (ws - 1.0), y_ctr + 0.5 * (hs - 1.0)])

    ratio_anchors = _mk(ws, hs, x_ctr, y_ctr)
    out = []
    for i in range(ratio_anchors.shape[0]):
        a = ratio_anchors[i]
        w = a[2] - a[0] + 1.0
        h = a[3] - a[1] + 1.0
        x_ctr = a[0] + 0.5 * (w - 1.0)
        y_ctr = a[1] + 0.5 * (h - 1.0)
        out.append(_mk(w * scales, h * scales, x_ctr, y_ctr))
    return np.vstack(out).astype(np.float32)


def _shift_anchors(shape, stride, anchors):
    sx = np.arange(shape[1]) * stride
    sy = np.arange(shape[0]) * stride
    SX, SY = np.meshgrid(sx, sy)
    shifts = np.stack([SX.ravel(), SY.ravel(), SX.ravel(), SY.ravel()], axis=1)
    return (anchors[None, :, :] + shifts[:, None, :]).reshape(-1, 4).astype(np.float32)


_ANCHORS_NP = _shift_anchors(_FEATURES_SHAPE, _STRIDE, _gen_anchors(_ANCHOR_SIZE))

_APAD = np.zeros((_NPAD, 4), dtype=np.float32)
_APAD[:, :] = np.array([0.0, 0.0, 15.0, 15.0], dtype=np.float32)
_APAD[:_N_REAL] = _ANCHORS_NP
_ABLK = (_APAD.reshape(_NW, _CHUNK, 4).transpose(0, 2, 1)
         .reshape(_NW, 4 * _CHUNK).copy())


def _splat_f(v):
    return jnp.full((_L,), v, jnp.float32)


def _splat_i(v):
    return jnp.full((_L,), v, jnp.int32)


def _log_f32(x):
    bits = lax.bitcast_convert_type(x, jnp.int32)
    e = lax.shift_right_logical(bits, _splat_i(23)) - _splat_i(127)
    mbits = lax.bitwise_or(lax.bitwise_and(bits, _splat_i(0x007FFFFF)),
                           _splat_i(0x3F800000))
    m = lax.bitcast_convert_type(mbits, jnp.float32)
    adj = m > _splat_f(_SQRT2)
    m = jnp.where(adj, m * 0.5, m)
    ef = e.astype(jnp.float32) + jnp.where(adj, _splat_f(1.0), _splat_f(0.0))
    s = (m - 1.0) / (m + 1.0)
    z = s * s
    p = z * (np.float32(1.0 / 3.0) + z * (np.float32(0.2) + z * (
        np.float32(1.0 / 7.0) + z * np.float32(1.0 / 9.0))))
    t2 = s + s
    return ef * np.float32(_LN2) + (t2 + t2 * p)


def _sc_body(raw_h, ablk_h, lab_h, dx_h, dy_h, dw_h, dh_h,
             raw_v, gtp_v, areab_v, ab_v, ob_v):
    wid = lax.axis_index("c") * _NS + lax.axis_index("s")

    pltpu.sync_copy(raw_h, raw_v)
    pltpu.sync_copy(ablk_h.at[wid], ab_v)

    def rep_step(j, jvec5):
        o = j * _L
        bx1 = plsc.load_gather(raw_v, [jvec5])
        by1 = plsc.load_gather(raw_v, [jvec5 + 1])
        bx2 = plsc.load_gather(raw_v, [jvec5 + 2])
        by2 = plsc.load_gather(raw_v, [jvec5 + 3])
        bcls = plsc.load_gather(raw_v, [jvec5 + 4])
        gtp_v[pl.ds(_OX1 + o, _L)] = bx1
        gtp_v[pl.ds(_OY1 + o, _L)] = by1
        gtp_v[pl.ds(_OX2 + o, _L)] = bx2
        gtp_v[pl.ds(_OY2 + o, _L)] = by2
        gtp_v[pl.ds(_OCLS + o, _L)] = bcls
        areab_v[pl.ds(o, _L)] = (bx2 - bx1 + 1.0) * (by2 - by1 + 1.0)
        return jvec5 + 5

    lax.fori_loop(0, _NUM_GT, rep_step, _splat_i(0))

    imh = plsc.load_gather(raw_v, [_splat_i(_OIMH_RAW)])
    imw = plsc.load_gather(raw_v, [_splat_i(_OIMW_RAW)])
    lane = lax.iota(jnp.int32, _L)

    def anchor_step(v, _):
        ax1 = [None] * _AUN
        ay1 = [None] * _AUN
        ax2 = [None] * _AUN
        ay2 = [None] * _AUN
        area_a = [None] * _AUN
        for k in range(_AUN):
            off = (v * _AUN + k) * _L
            ax1[k] = ab_v[pl.ds(off, _L)]
            ay1[k] = ab_v[pl.ds(_CHUNK + off, _L)]
            ax2[k] = ab_v[pl.ds(2 * _CHUNK + off, _L)]
            ay2[k] = ab_v[pl.ds(3 * _CHUNK + off, _L)]
            area_a[k] = (ax2[k] - ax1[k] + 1.0) * (ay2[k] - ay1[k] + 1.0)

        init = (tuple([_splat_f(-1.0)] * _AUN), tuple([_splat_i(0)] * _AUN),
                _splat_i(0))

        @plsc.parallel_loop(0, _NUM_GT, carry=init, unroll=2)
        def box_loop(j, carry):
            best, bidx, jvec = carry
            best = list(best)
            bidx = list(bidx)
            o = j * _L
            bx1 = gtp_v[pl.ds(_OX1 + o, _L)]
            by1 = gtp_v[pl.ds(_OY1 + o, _L)]
            bx2 = gtp_v[pl.ds(_OX2 + o, _L)]
            by2 = gtp_v[pl.ds(_OY2 + o, _L)]
            areab = areab_v[pl.ds(o, _L)]
            for k in range(_AUN):
                iw = jnp.maximum(
                    jnp.minimum(ax2[k], bx2) - jnp.maximum(ax1[k], bx1) + 1.0, 0.0)
                ih = jnp.maximum(
                    jnp.minimum(ay2[k], by2) - jnp.maximum(ay1[k], by1) + 1.0, 0.0)
                inter = iw * ih
                union = (area_a[k] + areab) - inter
                iou = inter / union
                upd = iou > best[k]
                best[k] = jnp.where(upd, iou, best[k])
                bidx[k] = jnp.where(upd, jvec, bidx[k])
            return tuple(best), tuple(bidx), jvec + 1

        best, bidx, _ = box_loop

        for k in range(_AUN):
            off = (v * _AUN + k) * _L
            gidx = bidx[k] * _L + lane
            bx1 = plsc.load_gather(gtp_v, [gidx + _OX1])
            by1 = plsc.load_gather(gtp_v, [gidx + _OY1])
            bx2 = plsc.load_gather(gtp_v, [gidx + _OX2])
            by2 = plsc.load_gather(gtp_v, [gidx + _OY2])
            bcls = plsc.load_gather(gtp_v, [gidx + _OCLS])

            lab = _splat_f(-1.0)
            lab = jnp.where(best[k] < _NEG_OVL, _splat_f(0.0), lab)
            lab = jnp.where(best[k] >= _POS_OVL, _splat_f(1.0), lab)
            inside = ((ax1[k] >= 0.0) & (ay1[k] >= 0.0)
                      & (ax2[k] < imw) & (ay2[k] < imh))
            lab = jnp.where(inside, lab, _splat_f(-1.0))
            lab = jnp.where(lab == 1.0, bcls, lab)
            ob_v[pl.ds(off, _L)] = lab

            ex_w = ax2[k] - ax1[k] + 1.0
            ex_h = ay2[k] - ay1[k] + 1.0
            gt_w = bx2 - bx1 + 1.0
            gt_h = by2 - by1 + 1.0
            ex_cx = ax1[k] + 0.5 * ex_w
            ex_cy = ay1[k] + 0.5 * ex_h
            gt_cx = bx1 + 0.5 * gt_w
            gt_cy = by1 + 0.5 * gt_h
            ob_v[pl.ds(_CHUNK + off, _L)] = (gt_cx - ex_cx) / ex_w
            ob_v[pl.ds(2 * _CHUNK + off, _L)] = (gt_cy - ex_cy) / ex_h
            ob_v[pl.ds(3 * _CHUNK + off, _L)] = _log_f32(gt_w / ex_w)
            ob_v[pl.ds(4 * _CHUNK + off, _L)] = _log_f32(gt_h / ex_h)
        return 0

    lax.fori_loop(0, _VPW // _AUN, anchor_step, 0)

    osl = pl.ds(wid * _CHUNK, _CHUNK)
    pltpu.sync_copy(ob_v.at[pl.ds(0, _CHUNK)], lab_h.at[osl])
    pltpu.sync_copy(ob_v.at[pl.ds(_CHUNK, _CHUNK)], dx_h.at[osl])
    pltpu.sync_copy(ob_v.at[pl.ds(2 * _CHUNK, _CHUNK)], dy_h.at[osl])
    pltpu.sync_copy(ob_v.at[pl.ds(3 * _CHUNK, _CHUNK)], dw_h.at[osl])
    pltpu.sync_copy(ob_v.at[pl.ds(4 * _CHUNK, _CHUNK)], dh_h.at[osl])


_sc_call = functools.partial(
    pl.kernel,
    out_type=[jax.ShapeDtypeStruct((_NPAD,), jnp.float32)] * 5,
    mesh=plsc.VectorSubcoreMesh(core_axis_name="c", subcore_axis_name="s",
                                num_cores=_NC, num_subcores=_NS),
    compiler_params=pltpu.CompilerParams(needs_layout_passes=False),
    scratch_types=[
        pltpu.VMEM((_RAW,), jnp.float32),
        pltpu.VMEM((5 * _GTLEN,), jnp.float32),
        pltpu.VMEM((_GTLEN,), jnp.float32),
        pltpu.VMEM((4 * _CHUNK,), jnp.float32),
        pltpu.VMEM((5 * _CHUNK,), jnp.float32),
    ],
)(_sc_body)


def kernel(im_info, gt_boxes):
    raw = jnp.concatenate([
        gt_boxes.astype(jnp.float32).reshape(_NUM_GT * 5),
        im_info.astype(jnp.float32).reshape(3),
        jnp.zeros((5,), jnp.float32),
    ])

    lab, dx, dy, dw, dh = _sc_call(raw, _ABLK)
    labels = lab[:_N_REAL][None, :]
    targets = jnp.stack([dx[:_N_REAL], dy[:_N_REAL],
                         dw[:_N_REAL], dh[:_N_REAL]], axis=1)[None]
    anchors = jnp.asarray(_ANCHORS_NP)[None]
    return labels, targets, anchors
